# baseline (speedup 1.0000x reference)
import jax
import jax.numpy as jnp
from jax import lax
from jax.experimental import pallas as pl
from jax.experimental.pallas import tpu as pltpu
from jax.experimental.pallas import tpu_sc as plsc

B = 4096
NR = 50
NW = 50
VOCAB = 100000
NSHAPE = 6
OUT = 128
SHAPE_E = 16
KERNEL_E = 16
BIAS_E = 2
NKERN = 8
REG_E = OUT - SHAPE_E
W_E = OUT - SHAPE_E - KERNEL_E - BIAS_E

N_NODES = B * NW
DW = SHAPE_E + KERNEL_E + BIAS_E

NC = 2
NS = 16
NWORK = NC * NS
NPAIR = B // 2
PPW = NPAIR // NWORK
NQUAD = PPW // 4

_M = 3200
_TB = 2000


def _dense_body(x_ref, wa_ref, wk2_ref, wb2_ref,
                bv_ref, swt_ref, sb_ref, dw_ref, dr_ref):
    x = x_ref[...].reshape(_M, 16)
    ws = x[:, 0:NSHAPE]
    wb = x[:, NSHAPE:NSHAPE + 2]
    kf = x[:, NSHAPE + 2:NSHAPE + 4]
    rs = x[:, NSHAPE + 4:16]
    i16 = lax.broadcasted_iota(jnp.int32, (_M, 2 * NKERN), 1)
    ksel = jnp.where(i16 < NKERN, kf[:, 0:1], kf[:, 1:2])
    oh = (ksel == (i16 & (NKERN - 1)).astype(jnp.float32)).astype(jnp.float32)
    dw = jnp.dot(ws, wa_ref[...], preferred_element_type=jnp.float32)
    dw += jnp.dot(oh, wk2_ref[...], preferred_element_type=jnp.float32)
    dw += jnp.dot(wb, wb2_ref[...], preferred_element_type=jnp.float32)
    dw_ref[...] = dw + bv_ref[...]
    dr_ref[...] = jnp.dot(rs, swt_ref[...],
                          preferred_element_type=jnp.float32) + sb_ref[...]


def _dense_parts(xpack, wa, wk2, wb2, bv, shape_WT, shape_b):
    grid = (N_NODES // _M,)
    row_spec = lambda w: pl.BlockSpec((_M, w), lambda i: (i, 0))
    full = lambda a: pl.BlockSpec(a.shape, lambda i: (0,) * a.ndim)
    return pl.pallas_call(
        _dense_body,
        grid=grid,
        in_specs=[
            pl.BlockSpec((_M // NW, NW, 16), lambda i: (i, 0, 0)),
            full(wa), full(wk2), full(wb2), full(bv),
            full(shape_WT), full(shape_b),
        ],
        out_specs=[row_spec(DW), row_spec(SHAPE_E)],
        out_shape=[
            jax.ShapeDtypeStruct((N_NODES, DW), jnp.float32),
            jax.ShapeDtypeStruct((N_NODES, SHAPE_E), jnp.float32),
        ],
    )(xpack, wa, wk2, wb2, bv, shape_WT, shape_b)


def _pad_body(wt_ref, rt_ref, ow_ref, or_ref):
    zw = jnp.zeros((_TB, OUT - W_E), jnp.float32)
    zr = jnp.zeros((_TB, OUT - REG_E), jnp.float32)
    ow_ref[...] = jnp.concatenate([wt_ref[...], zw], axis=1)
    or_ref[...] = jnp.concatenate([rt_ref[...], zr], axis=1)


def _pad_tables(w_table, r_table):
    grid = (VOCAB // _TB,)
    return pl.pallas_call(
        _pad_body,
        grid=grid,
        in_specs=[
            pl.BlockSpec((_TB, W_E), lambda i: (i, 0)),
            pl.BlockSpec((_TB, REG_E), lambda i: (i, 0)),
        ],
        out_specs=[
            pl.BlockSpec((_TB, OUT), lambda i: (i, 0)),
            pl.BlockSpec((_TB, OUT), lambda i: (i, 0)),
        ],
        out_shape=[
            jax.ShapeDtypeStruct((VOCAB, OUT), jnp.float32),
            jax.ShapeDtypeStruct((VOCAB, OUT), jnp.float32),
        ],
    )(w_table, r_table)


PAD_DW = DW - SHAPE_E
RING = 4
BPW = B // NWORK
NGRP = BPW // 8


def _sc_kernel(w_inds, r_inds, w_table, r_table, dense_w, dense_r, out,
               idxw, idxr, dbw, dbr, ob, gsem, osem):
    wid = lax.axis_index("s") * NC + lax.axis_index("c")
    bbase = wid * BPW

    def stage_idx(g, par):
        b8 = bbase + g * 8
        pltpu.sync_copy(w_inds.at[pl.ds(b8, 8)], idxw.at[par])
        pltpu.sync_copy(r_inds.at[pl.ds(b8, 8)], idxr.at[par])

    def stage_dense(g, h):
        n4 = (bbase + g * 8 + h * 4) * NW
        pltpu.sync_copy(dense_w.at[pl.ds(n4, 200)], dbw)
        pltpu.sync_copy(dense_r.at[pl.ds(n4, 200)], dbr)

    def gathers(par, i, buf):
        pltpu.async_copy(w_table.at[idxw.at[par, i]],
                         ob.at[buf, pl.ds(0, NW)], gsem.at[buf])
        pltpu.async_copy(r_table.at[idxr.at[par, i]],
                         ob.at[buf, pl.ds(NW, NR)], gsem.at[buf])

    def wait_gathers(par, i, buf):
        pltpu.make_async_copy(w_table.at[idxw.at[par, i]],
                              ob.at[buf, pl.ds(0, NW)], gsem.at[buf]).wait()
        pltpu.make_async_copy(r_table.at[idxr.at[par, i]],
                              ob.at[buf, pl.ds(NW, NR)], gsem.at[buf]).wait()

    def merge(ih, buf):
        def mbody(r, _):
            dn = ih * NW + r
            ob[buf, r, pl.ds(W_E, 16)] = dbw[dn, pl.ds(0, 16)]
            ob[buf, r, pl.ds(W_E + 16, 16)] = dbw[dn, pl.ds(16, 16)]
            ob[buf, r, pl.ds(REG_E, 16)] = dbw[dn, pl.ds(PAD_DW, 16)]
            ob[buf, NW + r, pl.ds(REG_E, 16)] = dbr[dn, pl.ds(0, 16)]
            return 0

        lax.fori_loop(0, NW, mbody, 0)

    def out_issue(b, buf):
        pltpu.async_copy(ob.at[buf], out.at[b], osem.at[buf])

    def wait_out(b, buf):
        pltpu.make_async_copy(ob.at[buf], out.at[b], osem.at[buf]).wait()

    stage_idx(0, 0)
    gathers(0, 0, 0)
    gathers(0, 1, 1)

    def group(g, par):
        for h in range(2):
            stage_dense(g, h)
            if h == 1:
                @pl.when(g < NGRP - 1)
                def _():
                    stage_idx(g + 1, 1 - par)
            for ih in range(4):
                i = h * 4 + ih
                b = bbase + g * 8 + i
                buf = i % RING
                wait_gathers(par, i, buf)
                merge(ih, buf)
                out_issue(b, buf)
                nbuf = (i + 2) % RING
                ob_prev = i + 2 - RING
                if ob_prev >= 0:
                    wait_out(b - 2, nbuf)
                else:
                    @pl.when(g > 0)
                    def _():
                        wait_out(b - 2, nbuf)
                if i < 6:
                    gathers(par, i + 2, nbuf)
                else:
                    @pl.when(g < NGRP - 1)
                    def _():
                        gathers(1 - par, i + 2 - 8, nbuf)

    def body(k2, _):
        group(2 * k2, 0)
        group(2 * k2 + 1, 1)
        return 0

    lax.fori_loop(0, NGRP // 2, body, 0)
    wait_out(bbase + BPW - 2, 6 % RING)
    wait_out(bbase + BPW - 1, 7 % RING)


def _sc_gather_assemble(w_inds, r_inds, w_table, r_table, dense_w, dense_r):
    mesh = plsc.VectorSubcoreMesh(core_axis_name="c", subcore_axis_name="s")
    f = pl.kernel(
        _sc_kernel,
        out_type=jax.ShapeDtypeStruct((B, NW + NR, OUT), jnp.float32),
        mesh=mesh,
        scratch_types=[
            pltpu.VMEM((2, 8, NW), jnp.int32),
            pltpu.VMEM((2, 8, NR), jnp.int32),
            pltpu.VMEM((200, DW), jnp.float32),
            pltpu.VMEM((200, SHAPE_E), jnp.float32),
            pltpu.VMEM((RING, NW + NR, OUT), jnp.float32),
            pltpu.SemaphoreType.DMA((RING,)),
            pltpu.SemaphoreType.DMA((RING,)),
        ],
    )
    return f(w_inds, r_inds, w_table, r_table, dense_w, dense_r)


def kernel(regular_node_inds, regular_node_shapes, weighted_node_inds,
           weighted_node_shapes, weighted_node_kernels, weighted_node_bias,
           regular_table, weighted_table, kernel_table,
           shape_W, shape_b, bias_W, bias_b):
    shape_WT = shape_W.T
    z = jnp.zeros
    f32 = jnp.float32
    wa = jnp.concatenate([shape_WT, z((NSHAPE, DW - SHAPE_E), f32)], axis=1)
    ktd = jnp.concatenate([
        jnp.concatenate([kernel_table, z((NKERN, NKERN), f32)], axis=1),
        jnp.concatenate([z((NKERN, NKERN), f32), kernel_table], axis=1),
    ], axis=0)
    wk2 = jnp.concatenate(
        [z((2 * NKERN, SHAPE_E), f32), ktd, z((2 * NKERN, BIAS_E), f32)],
        axis=1)
    wb2 = jnp.concatenate([z((2, DW - BIAS_E), f32), bias_W.T], axis=1)
    bv = jnp.concatenate(
        [shape_b, z((KERNEL_E,), f32), bias_b]).reshape(1, DW)
    xpack = jnp.concatenate(
        [weighted_node_shapes, weighted_node_bias,
         weighted_node_kernels.astype(f32), regular_node_shapes],
        axis=-1)
    dense_w, dense_r = _dense_parts(
        xpack, wa, wk2, wb2, bv, shape_WT, shape_b.reshape(1, SHAPE_E))
    w_table_p, r_table_p = _pad_tables(weighted_table, regular_table)
    return _sc_gather_assemble(weighted_node_inds, regular_node_inds,
                               w_table_p, r_table_p, dense_w, dense_r)


# speedup vs baseline: 1.5260x; 1.5260x over previous
import jax
import jax.numpy as jnp
from jax import lax
from jax.experimental import pallas as pl
from jax.experimental.pallas import tpu as pltpu
from jax.experimental.pallas import tpu_sc as plsc

B = 4096
NR = 50
NW = 50
VOCAB = 100000
NSHAPE = 6
OUT = 128
SHAPE_E = 16
KERNEL_E = 16
BIAS_E = 2
NKERN = 8
REG_E = OUT - SHAPE_E
W_E = OUT - SHAPE_E - KERNEL_E - BIAS_E

N_NODES = B * NW
DW = SHAPE_E + KERNEL_E + BIAS_E

NC = 2
NS = 16
NWORK = NC * NS
NPAIR = B // 2
PPW = NPAIR // NWORK
NQUAD = PPW // 4

_M = 6400
_TB = 2000


def _dense_body(x_ref, wa_ref, wk2_ref, wb2_ref,
                bv_ref, swt_ref, sb_ref, dw_ref, dr_ref):
    x = x_ref[...]
    ws = x[:, 0:NSHAPE]
    wb = x[:, NSHAPE:NSHAPE + 2]
    kf = x[:, NSHAPE + 2:NSHAPE + 4]
    rs = x[:, NSHAPE + 4:16]
    i16 = lax.broadcasted_iota(jnp.int32, (_M, 2 * NKERN), 1)
    ksel = jnp.where(i16 < NKERN, kf[:, 0:1], kf[:, 1:2])
    oh = (ksel == (i16 & (NKERN - 1)).astype(jnp.float32)).astype(jnp.float32)
    dw = jnp.dot(ws, wa_ref[...], preferred_element_type=jnp.float32)
    dw += jnp.dot(oh, wk2_ref[...], preferred_element_type=jnp.float32)
    dw += jnp.dot(wb, wb2_ref[...], preferred_element_type=jnp.float32)
    dw_ref[...] = dw + bv_ref[...]
    dr_ref[...] = jnp.dot(rs, swt_ref[...],
                          preferred_element_type=jnp.float32) + sb_ref[...]


def _dense_parts(xpack, wa, wk2, wb2, bv, shape_WT, shape_b):
    grid = (N_NODES // _M,)
    row_spec = lambda w: pl.BlockSpec((_M, w), lambda i: (i, 0))
    full = lambda a: pl.BlockSpec(a.shape, lambda i: (0,) * a.ndim)
    return pl.pallas_call(
        _dense_body,
        grid=grid,
        in_specs=[
            pl.BlockSpec((_M, 16), lambda i: (i, 0)),
            full(wa), full(wk2), full(wb2), full(bv),
            full(shape_WT), full(shape_b),
        ],
        out_specs=[row_spec(DW), row_spec(SHAPE_E)],
        out_shape=[
            jax.ShapeDtypeStruct((N_NODES, DW), jnp.float32),
            jax.ShapeDtypeStruct((N_NODES, SHAPE_E), jnp.float32),
        ],
    )(xpack, wa, wk2, wb2, bv, shape_WT, shape_b)


def _pad_body(wt_ref, rt_ref, ow_ref, or_ref):
    zw = jnp.zeros((_TB, OUT - W_E), jnp.float32)
    zr = jnp.zeros((_TB, OUT - REG_E), jnp.float32)
    ow_ref[...] = jnp.concatenate([wt_ref[...], zw], axis=1)
    or_ref[...] = jnp.concatenate([rt_ref[...], zr], axis=1)


def _pad_tables(w_table, r_table):
    grid = (VOCAB // _TB,)
    return pl.pallas_call(
        _pad_body,
        grid=grid,
        in_specs=[
            pl.BlockSpec((_TB, W_E), lambda i: (i, 0)),
            pl.BlockSpec((_TB, REG_E), lambda i: (i, 0)),
        ],
        out_specs=[
            pl.BlockSpec((_TB, OUT), lambda i: (i, 0)),
            pl.BlockSpec((_TB, OUT), lambda i: (i, 0)),
        ],
        out_shape=[
            jax.ShapeDtypeStruct((VOCAB, OUT), jnp.float32),
            jax.ShapeDtypeStruct((VOCAB, OUT), jnp.float32),
        ],
    )(w_table, r_table)


PAD_DW = DW - SHAPE_E
RING = 4
BPW = B // NWORK
NGRP = BPW // 8


def _sc_kernel(w_inds, r_inds, w_table, r_table, dense_w, dense_r, out,
               idxw, idxr, dbw, dbr, ob, gsem, osem):
    wid = lax.axis_index("s") * NC + lax.axis_index("c")
    bbase = wid * BPW

    def stage_idx(g, par):
        b8 = bbase + g * 8
        pltpu.sync_copy(w_inds.at[pl.ds(b8, 8)], idxw.at[par])
        pltpu.sync_copy(r_inds.at[pl.ds(b8, 8)], idxr.at[par])

    def stage_dense(g, h):
        n4 = (bbase + g * 8 + h * 4) * NW
        pltpu.sync_copy(dense_w.at[pl.ds(n4, 200)], dbw)
        pltpu.sync_copy(dense_r.at[pl.ds(n4, 200)], dbr)

    def gathers(par, i, buf):
        pltpu.async_copy(w_table.at[idxw.at[par, i]],
                         ob.at[buf, pl.ds(0, NW)], gsem.at[buf])
        pltpu.async_copy(r_table.at[idxr.at[par, i]],
                         ob.at[buf, pl.ds(NW, NR)], gsem.at[buf])

    def wait_gathers(par, i, buf):
        pltpu.make_async_copy(w_table.at[idxw.at[par, i]],
                              ob.at[buf, pl.ds(0, NW)], gsem.at[buf]).wait()
        pltpu.make_async_copy(r_table.at[idxr.at[par, i]],
                              ob.at[buf, pl.ds(NW, NR)], gsem.at[buf]).wait()

    def merge(ih, buf):
        def mbody(r, _):
            dn = ih * NW + r
            ob[buf, r, pl.ds(W_E, 16)] = dbw[dn, pl.ds(0, 16)]
            ob[buf, r, pl.ds(W_E + 16, 16)] = dbw[dn, pl.ds(16, 16)]
            ob[buf, r, pl.ds(REG_E, 16)] = dbw[dn, pl.ds(PAD_DW, 16)]
            ob[buf, NW + r, pl.ds(REG_E, 16)] = dbr[dn, pl.ds(0, 16)]
            return 0

        lax.fori_loop(0, NW, mbody, 0)

    def out_issue(b, buf):
        pltpu.async_copy(ob.at[buf], out.at[b], osem.at[buf])

    def wait_out(b, buf):
        pltpu.make_async_copy(ob.at[buf], out.at[b], osem.at[buf]).wait()

    stage_idx(0, 0)
    gathers(0, 0, 0)
    gathers(0, 1, 1)

    def group(g, par):
        for h in range(2):
            stage_dense(g, h)
            if h == 1:
                @pl.when(g < NGRP - 1)
                def _():
                    stage_idx(g + 1, 1 - par)
            for ih in range(4):
                i = h * 4 + ih
                b = bbase + g * 8 + i
                buf = i % RING
                wait_gathers(par, i, buf)
                merge(ih, buf)
                out_issue(b, buf)
                nbuf = (i + 2) % RING
                ob_prev = i + 2 - RING
                if ob_prev >= 0:
                    wait_out(b - 2, nbuf)
                else:
                    @pl.when(g > 0)
                    def _():
                        wait_out(b - 2, nbuf)
                if i < 6:
                    gathers(par, i + 2, nbuf)
                else:
                    @pl.when(g < NGRP - 1)
                    def _():
                        gathers(1 - par, i + 2 - 8, nbuf)

    def body(k2, _):
        group(2 * k2, 0)
        group(2 * k2 + 1, 1)
        return 0

    lax.fori_loop(0, NGRP // 2, body, 0)
    wait_out(bbase + BPW - 2, 6 % RING)
    wait_out(bbase + BPW - 1, 7 % RING)


def _sc_gather_assemble(w_inds, r_inds, w_table, r_table, dense_w, dense_r):
    mesh = plsc.VectorSubcoreMesh(core_axis_name="c", subcore_axis_name="s")
    f = pl.kernel(
        _sc_kernel,
        out_type=jax.ShapeDtypeStruct((B, NW + NR, OUT), jnp.float32),
        mesh=mesh,
        scratch_types=[
            pltpu.VMEM((2, 8, NW), jnp.int32),
            pltpu.VMEM((2, 8, NR), jnp.int32),
            pltpu.VMEM((200, DW), jnp.float32),
            pltpu.VMEM((200, SHAPE_E), jnp.float32),
            pltpu.VMEM((RING, NW + NR, OUT), jnp.float32),
            pltpu.SemaphoreType.DMA((RING,)),
            pltpu.SemaphoreType.DMA((RING,)),
        ],
    )
    return f(w_inds, r_inds, w_table, r_table, dense_w, dense_r)


def kernel(regular_node_inds, regular_node_shapes, weighted_node_inds,
           weighted_node_shapes, weighted_node_kernels, weighted_node_bias,
           regular_table, weighted_table, kernel_table,
           shape_W, shape_b, bias_W, bias_b):
    shape_WT = shape_W.T
    z = jnp.zeros
    f32 = jnp.float32
    wa = jnp.concatenate([shape_WT, z((NSHAPE, DW - SHAPE_E), f32)], axis=1)
    ktd = jnp.concatenate([
        jnp.concatenate([kernel_table, z((NKERN, NKERN), f32)], axis=1),
        jnp.concatenate([z((NKERN, NKERN), f32), kernel_table], axis=1),
    ], axis=0)
    wk2 = jnp.concatenate(
        [z((2 * NKERN, SHAPE_E), f32), ktd, z((2 * NKERN, BIAS_E), f32)],
        axis=1)
    wb2 = jnp.concatenate([z((2, DW - BIAS_E), f32), bias_W.T], axis=1)
    bv = jnp.concatenate(
        [shape_b, z((KERNEL_E,), f32), bias_b]).reshape(1, DW)
    xpack = jnp.concatenate(
        [weighted_node_shapes, weighted_node_bias,
         weighted_node_kernels.astype(f32), regular_node_shapes],
        axis=-1).reshape(N_NODES, 16)
    dense_w, dense_r = _dense_parts(
        xpack, wa, wk2, wb2, bv, shape_WT, shape_b.reshape(1, SHAPE_E))
    w_table_p, r_table_p = _pad_tables(weighted_table, regular_table)
    return _sc_gather_assemble(weighted_node_inds, regular_node_inds,
                               w_table_p, r_table_p, dense_w, dense_r)


# speedup vs baseline: 1.5263x; 1.0001x over previous
import jax
import jax.numpy as jnp
from jax import lax
from jax.experimental import pallas as pl
from jax.experimental.pallas import tpu as pltpu
from jax.experimental.pallas import tpu_sc as plsc

B = 4096
NR = 50
NW = 50
VOCAB = 100000
NSHAPE = 6
OUT = 128
SHAPE_E = 16
KERNEL_E = 16
BIAS_E = 2
NKERN = 8
REG_E = OUT - SHAPE_E
W_E = OUT - SHAPE_E - KERNEL_E - BIAS_E

N_NODES = B * NW
DW = SHAPE_E + KERNEL_E + BIAS_E

NC = 2
NS = 16
NWORK = NC * NS

_M = 6400
_TB = 2000


def _dense_body(x_ref, wa_ref, wk2_ref, wb2_ref,
                bv_ref, swt_ref, sb_ref, dw_ref, dr_ref):
    x = x_ref[...]
    ws = x[:, 0:NSHAPE]
    wb = x[:, NSHAPE:NSHAPE + 2]
    kf = x[:, NSHAPE + 2:NSHAPE + 4]
    rs = x[:, NSHAPE + 4:16]
    i16 = lax.broadcasted_iota(jnp.int32, (_M, 2 * NKERN), 1)
    ksel = jnp.where(i16 < NKERN, kf[:, 0:1], kf[:, 1:2])
    oh = (ksel == (i16 & (NKERN - 1)).astype(jnp.float32)).astype(jnp.float32)
    dw = jnp.dot(ws, wa_ref[...], preferred_element_type=jnp.float32)
    dw += jnp.dot(oh, wk2_ref[...], preferred_element_type=jnp.float32)
    dw += jnp.dot(wb, wb2_ref[...], preferred_element_type=jnp.float32)
    dw_ref[...] = dw + bv_ref[...]
    dr_ref[...] = jnp.dot(rs, swt_ref[...],
                          preferred_element_type=jnp.float32) + sb_ref[...]


def _dense_parts(xpack, wa, wk2, wb2, bv, shape_WT, shape_b):
    grid = (N_NODES // _M,)
    row_spec = lambda w: pl.BlockSpec((_M, w), lambda i: (i, 0))
    full = lambda a: pl.BlockSpec(a.shape, lambda i: (0,) * a.ndim)
    return pl.pallas_call(
        _dense_body,
        grid=grid,
        in_specs=[
            pl.BlockSpec((_M, 16), lambda i: (i, 0)),
            full(wa), full(wk2), full(wb2), full(bv),
            full(shape_WT), full(shape_b),
        ],
        out_specs=[row_spec(DW), row_spec(SHAPE_E)],
        out_shape=[
            jax.ShapeDtypeStruct((N_NODES, DW), jnp.float32),
            jax.ShapeDtypeStruct((N_NODES, SHAPE_E), jnp.float32),
        ],
    )(xpack, wa, wk2, wb2, bv, shape_WT, shape_b)


def _pad_body(wt_ref, rt_ref, ow_ref, or_ref):
    zw = jnp.zeros((_TB, OUT - W_E), jnp.float32)
    zr = jnp.zeros((_TB, OUT - REG_E), jnp.float32)
    ow_ref[...] = jnp.concatenate([wt_ref[...], zw], axis=1)
    or_ref[...] = jnp.concatenate([rt_ref[...], zr], axis=1)


def _pad_tables(w_table, r_table):
    grid = (VOCAB // _TB,)
    return pl.pallas_call(
        _pad_body,
        grid=grid,
        in_specs=[
            pl.BlockSpec((_TB, W_E), lambda i: (i, 0)),
            pl.BlockSpec((_TB, REG_E), lambda i: (i, 0)),
        ],
        out_specs=[
            pl.BlockSpec((_TB, OUT), lambda i: (i, 0)),
            pl.BlockSpec((_TB, OUT), lambda i: (i, 0)),
        ],
        out_shape=[
            jax.ShapeDtypeStruct((VOCAB, OUT), jnp.float32),
            jax.ShapeDtypeStruct((VOCAB, OUT), jnp.float32),
        ],
    )(w_table, r_table)


PAD_DW = DW - SHAPE_E
RING = 4
BPW = B // NWORK
NGRP = BPW // 8


def _sc_kernel(w_inds, r_inds, w_table, r_table, dense_w, dense_r, out,
               idxw, idxr, dbw, dbr, ob, gsem, osem):
    wid = lax.axis_index("s") * NC + lax.axis_index("c")
    bbase = wid * BPW

    def stage_idx(g, par):
        b8 = bbase + g * 8
        pltpu.sync_copy(w_inds.at[pl.ds(b8, 8)], idxw.at[par])
        pltpu.sync_copy(r_inds.at[pl.ds(b8, 8)], idxr.at[par])

    def stage_dense(g, h):
        n4 = (bbase + g * 8 + h * 4) * NW
        pltpu.sync_copy(dense_w.at[pl.ds(n4, 200)], dbw)
        pltpu.sync_copy(dense_r.at[pl.ds(n4, 200)], dbr)

    def gathers(par, i, buf):
        pltpu.async_copy(w_table.at[idxw.at[par, i]],
                         ob.at[buf, pl.ds(0, NW)], gsem.at[buf])
        pltpu.async_copy(r_table.at[idxr.at[par, i]],
                         ob.at[buf, pl.ds(NW, NR)], gsem.at[buf])

    def wait_gathers(par, i, buf):
        pltpu.make_async_copy(w_table.at[idxw.at[par, i]],
                              ob.at[buf, pl.ds(0, NW)], gsem.at[buf]).wait()
        pltpu.make_async_copy(r_table.at[idxr.at[par, i]],
                              ob.at[buf, pl.ds(NW, NR)], gsem.at[buf]).wait()

    def merge(ih, buf):
        def mbody(r, _):
            dn = ih * NW + r
            ob[buf, r, pl.ds(W_E, 16)] = dbw[dn, pl.ds(0, 16)]
            ob[buf, r, pl.ds(W_E + 16, 16)] = dbw[dn, pl.ds(16, 16)]
            ob[buf, r, pl.ds(REG_E, 16)] = dbw[dn, pl.ds(PAD_DW, 16)]
            ob[buf, NW + r, pl.ds(REG_E, 16)] = dbr[dn, pl.ds(0, 16)]
            return 0

        lax.fori_loop(0, NW, mbody, 0)

    def out_issue(b, buf):
        pltpu.async_copy(ob.at[buf], out.at[b], osem.at[buf])

    def wait_out(b, buf):
        pltpu.make_async_copy(ob.at[buf], out.at[b], osem.at[buf]).wait()

    stage_idx(0, 0)
    gathers(0, 0, 0)
    gathers(0, 1, 1)

    def group(g, par):
        for h in range(2):
            stage_dense(g, h)
            if h == 1:
                @pl.when(g < NGRP - 1)
                def _():
                    stage_idx(g + 1, 1 - par)
            for ih in range(4):
                i = h * 4 + ih
                b = bbase + g * 8 + i
                buf = i % RING
                wait_gathers(par, i, buf)
                merge(ih, buf)
                out_issue(b, buf)
                nbuf = (i + 2) % RING
                ob_prev = i + 2 - RING
                if ob_prev >= 0:
                    wait_out(b - 2, nbuf)
                else:
                    @pl.when(g > 0)
                    def _():
                        wait_out(b - 2, nbuf)
                if i < 6:
                    gathers(par, i + 2, nbuf)
                else:
                    @pl.when(g < NGRP - 1)
                    def _():
                        gathers(1 - par, i + 2 - 8, nbuf)

    def body(k2, _):
        group(2 * k2, 0)
        group(2 * k2 + 1, 1)
        return 0

    lax.fori_loop(0, NGRP // 2, body, 0)
    wait_out(bbase + BPW - 2, 6 % RING)
    wait_out(bbase + BPW - 1, 7 % RING)


def _sc_gather_assemble(w_inds, r_inds, w_table, r_table, dense_w, dense_r):
    mesh = plsc.VectorSubcoreMesh(core_axis_name="c", subcore_axis_name="s")
    f = pl.kernel(
        _sc_kernel,
        out_type=jax.ShapeDtypeStruct((B, NW + NR, OUT), jnp.float32),
        mesh=mesh,
        scratch_types=[
            pltpu.VMEM((2, 8, NW), jnp.int32),
            pltpu.VMEM((2, 8, NR), jnp.int32),
            pltpu.VMEM((200, DW), jnp.float32),
            pltpu.VMEM((200, SHAPE_E), jnp.float32),
            pltpu.VMEM((RING, NW + NR, OUT), jnp.float32),
            pltpu.SemaphoreType.DMA((RING,)),
            pltpu.SemaphoreType.DMA((RING,)),
        ],
    )
    return f(w_inds, r_inds, w_table, r_table, dense_w, dense_r)


def kernel(regular_node_inds, regular_node_shapes, weighted_node_inds,
           weighted_node_shapes, weighted_node_kernels, weighted_node_bias,
           regular_table, weighted_table, kernel_table,
           shape_W, shape_b, bias_W, bias_b):
    shape_WT = shape_W.T
    z = jnp.zeros
    f32 = jnp.float32
    wa = jnp.concatenate([shape_WT, z((NSHAPE, DW - SHAPE_E), f32)], axis=1)
    ktd = jnp.concatenate([
        jnp.concatenate([kernel_table, z((NKERN, NKERN), f32)], axis=1),
        jnp.concatenate([z((NKERN, NKERN), f32), kernel_table], axis=1),
    ], axis=0)
    wk2 = jnp.concatenate(
        [z((2 * NKERN, SHAPE_E), f32), ktd, z((2 * NKERN, BIAS_E), f32)],
        axis=1)
    wb2 = jnp.concatenate([z((2, DW - BIAS_E), f32), bias_W.T], axis=1)
    bv = jnp.concatenate(
        [shape_b, z((KERNEL_E,), f32), bias_b]).reshape(1, DW)
    xpack = jnp.concatenate(
        [weighted_node_shapes, weighted_node_bias,
         weighted_node_kernels.astype(f32), regular_node_shapes],
        axis=-1).reshape(N_NODES, 16)
    dense_w, dense_r = _dense_parts(
        xpack, wa, wk2, wb2, bv, shape_WT, shape_b.reshape(1, SHAPE_E))
    w_table_p, r_table_p = _pad_tables(weighted_table, regular_table)
    return _sc_gather_assemble(weighted_node_inds, regular_node_inds,
                               w_table_p, r_table_p, dense_w, dense_r)
